# Initial kernel scaffold; baseline (speedup 1.0000x reference)
import jax
import jax.numpy as jnp
from jax.experimental import pallas as pl


def kernel(input_ids, word_embeddings, position_embeddings):
    raise NotImplementedError("write your pallas kernel here")


# speedup vs baseline: 1.7036x; 1.7036x over previous
import functools

import jax
import jax.numpy as jnp
from jax import lax
from jax.experimental import pallas as pl
from jax.experimental.pallas import tpu as pltpu
from jax.experimental.pallas import tpu_sc as plsc

VOCAB = 100000
HIDDEN = 768
MAX_POS = 512
BATCH = 1024
SEQ = 512

B = BATCH * SEQ
NC, NS = 2, 16
NW = NC * NS
BPW = B // NW
SPW = BPW // SEQ
C = 32
NPH = SEQ // C
CPW = BPW // C
NVEC = HIDDEN // 16


def _body(ids_hbm, word_hbm, pos_hbm, out_hbm, idx_v, pos_v, word_v, sem):
    wid = lax.axis_index("s") * NC + lax.axis_index("c")
    pltpu.sync_copy(ids_hbm.at[wid], idx_v)

    def add_row(r, _):
        for j in range(NVEC):
            sl = pl.ds(j * 16, 16)
            word_v[r, sl] = word_v[r, sl] + pos_v[r, sl]
        return _

    def seq_loop(s, p):
        chunk = s * NPH + p
        pltpu.async_copy(word_hbm.at[idx_v.at[chunk]], word_v, sem).wait()
        lax.fori_loop(0, C, add_row, 0, unroll=2)
        row0 = wid * BPW + s * SEQ + p * C
        pltpu.sync_copy(word_v, out_hbm.at[pl.ds(row0, C)])
        return p

    def phase_loop(p, _):
        pltpu.sync_copy(pos_hbm.at[pl.ds(p * C, C)], pos_v)
        lax.fori_loop(0, SPW, seq_loop, p)
        return _

    lax.fori_loop(0, NPH, phase_loop, 0)


@functools.partial(jax.jit, static_argnums=())
def _run(ids3, word_embeddings, position_embeddings):
    mesh = plsc.VectorSubcoreMesh(core_axis_name="c", subcore_axis_name="s")
    f = functools.partial(
        pl.kernel,
        mesh=mesh,
        out_type=jax.ShapeDtypeStruct((B, HIDDEN), jnp.float32),
        scratch_types=[
            pltpu.VMEM((CPW, C), jnp.int32),
            pltpu.VMEM((C, HIDDEN), jnp.float32),
            pltpu.VMEM((C, HIDDEN), jnp.float32),
            pltpu.SemaphoreType.DMA,
        ],
    )(_body)
    return f(ids3, word_embeddings, position_embeddings)


def kernel(input_ids, word_embeddings, position_embeddings):
    ids3 = input_ids.astype(jnp.int32).reshape(NW, CPW, C)
    out = _run(ids3, word_embeddings, position_embeddings)
    return out.reshape(BATCH, SEQ, HIDDEN)
